# baseline (speedup 1.0000x reference)
import functools

import jax
import jax.numpy as jnp
from jax import lax
from jax.experimental import pallas as pl
from jax.experimental.pallas import tpu as pltpu

B, E, V, D = 64, 100, 50, 768


def _fused_kernel(ent_ref, sr_ref, ev_ref, emask_ref, vmask_ref,
                  elab_ref, vlab_ref, wa_ref, ba_ref, we_ref, be_ref,
                  out_ref, ua_scr, ue_scr):
    b = pl.program_id(0)

    @pl.when(b == 0)
    def _init():
        sr = sr_ref[...]
        ua_scr[...] = lax.dot_general(
            sr, wa_ref[0], (((1,), (1,)), ((), ())),
            preferred_element_type=jnp.float32)
        ue_scr[...] = lax.dot_general(
            sr, we_ref[0], (((1,), (1,)), ((), ())),
            preferred_element_type=jnp.float32)
        out_ref[...] = jnp.zeros((1, 1), jnp.float32)

    ua = ua_scr[pl.ds(b, 1), :]
    ue = ue_scr[pl.ds(b, 1), :]

    za = lax.dot_general(ua, ent_ref[0], (((1,), (1,)), ((), ())),
                         preferred_element_type=jnp.float32)
    zv = lax.dot_general(ue, ev_ref[0], (((1,), (1,)), ((), ())),
                         preferred_element_type=jnp.float32)

    za = (za + ba_ref[0]) * emask_ref[pl.ds(b, 1), :]
    zv = (zv + be_ref[0]) * vmask_ref[pl.ds(b, 1), :]

    ya = elab_ref[pl.ds(b, 1), :].astype(jnp.float32)
    yv = vlab_ref[pl.ds(b, 1), :].astype(jnp.float32)

    bce_a = jnp.maximum(za, 0.0) - za * ya + jnp.log1p(jnp.exp(-jnp.abs(za)))
    bce_v = jnp.maximum(zv, 0.0) - zv * yv + jnp.log1p(jnp.exp(-jnp.abs(zv)))

    sa = jnp.sum(bce_a, axis=1, keepdims=True)
    sv = jnp.sum(bce_v, axis=1, keepdims=True)
    out_ref[...] += (0.5 / (B * E)) * sa + (0.5 / (B * V)) * sv


@functools.partial(jax.jit, static_argnames=())
def kernel(entity_mat, sr_vec, ev_mat, entity_mask, evidence_mask,
           entity_labels, evidence_labels, W_answer, b_answer,
           W_evidence, b_evidence):
    grid = (B,)
    whole = lambda b: (0, 0)
    out = pl.pallas_call(
        _fused_kernel,
        grid=grid,
        in_specs=[
            pl.BlockSpec((1, E, D), lambda b: (b, 0, 0)),
            pl.BlockSpec((B, D), whole),
            pl.BlockSpec((1, V, D), lambda b: (b, 0, 0)),
            pl.BlockSpec((B, E), whole),
            pl.BlockSpec((B, V), whole),
            pl.BlockSpec((B, E), whole),
            pl.BlockSpec((B, V), whole),
            pl.BlockSpec((1, D, D), lambda b: (0, 0, 0)),
            pl.BlockSpec(memory_space=pltpu.SMEM),
            pl.BlockSpec((1, D, D), lambda b: (0, 0, 0)),
            pl.BlockSpec(memory_space=pltpu.SMEM),
        ],
        out_specs=pl.BlockSpec((1, 1), whole),
        out_shape=jax.ShapeDtypeStruct((1, 1), jnp.float32),
        scratch_shapes=[
            pltpu.VMEM((B, D), jnp.float32),
            pltpu.VMEM((B, D), jnp.float32),
        ],
    )(entity_mat, sr_vec, ev_mat, entity_mask, evidence_mask,
      entity_labels, evidence_labels, W_answer, b_answer,
      W_evidence, b_evidence)
    return out[0, 0]
